# baseline (speedup 1.0000x reference)
import functools

import jax
import jax.numpy as jnp
from jax import lax
from jax.experimental import pallas as pl
from jax.experimental.pallas import tpu as pltpu
from jax.experimental.pallas import tpu_sc as plsc

_TOP_RATIO = 0.3
_TOP_WEIGHT = 1.0
_IGNORE_INDEX = -1

_VB = 2048


def _sc_gather(pred_flat, gt, n, c):
    info = plsc.get_sparse_core_info()
    nc, ns, lanes = info.num_cores, info.num_subcores, info.num_lanes
    nw = nc * ns
    assert n % (8 * nw) == 0
    b_per_w = n // nw
    mesh = plsc.VectorSubcoreMesh(core_axis_name="c", subcore_axis_name="s")

    @functools.partial(
        pl.kernel,
        mesh=mesh,
        out_type=jax.ShapeDtypeStruct((n,), jnp.float32),
        scratch_types=[
            pltpu.VMEM((b_per_w,), jnp.int32),
            pltpu.VMEM((b_per_w,), jnp.int32),
            pltpu.VMEM((b_per_w,), jnp.float32),
            pltpu.SemaphoreType.DMA,
        ],
    )
    def gather_k(gt_hbm, pred_hbm, out_hbm, gt_v, flat_v, vals_v, sem):
        wid = lax.axis_index("s") * nc + lax.axis_index("c")
        base = wid * b_per_w
        pltpu.sync_copy(gt_hbm.at[pl.ds(base, b_per_w)], gt_v)
        for i in range(b_per_w // lanes):
            g = gt_v[pl.ds(i * lanes, lanes)]
            g = jnp.minimum(jnp.maximum(g, 0), c - 1)
            rows = base + i * lanes + lax.iota(jnp.int32, lanes)
            flat_v[pl.ds(i * lanes, lanes)] = rows * c + g
        pltpu.async_copy(pred_hbm.at[flat_v], vals_v, sem).wait()
        pltpu.sync_copy(vals_v, out_hbm.at[pl.ds(base, b_per_w)])

    return gather_k(gt, pred_flat)


def _lse_body(c, pred_ref, lse_ref, m_ref, s_ref):
    j = pl.program_id(0)
    nb = pl.num_programs(0)
    n, vb = pred_ref.shape
    x = pred_ref[...]

    def _mask(x):
        cols = j * vb + lax.broadcasted_iota(jnp.int32, (n, vb), 1)
        return jnp.where(cols < c, x, -jnp.inf)

    x = lax.cond(j == nb - 1, _mask, lambda x: x, x)

    @pl.when(j == 0)
    def _():
        m_ref[...] = jnp.full((n, 1), -jnp.inf, jnp.float32)
        s_ref[...] = jnp.zeros((n, 1), jnp.float32)

    bm = jnp.max(x, axis=1, keepdims=True)
    m_old = m_ref[...]
    m_new = jnp.maximum(m_old, bm)
    y = jnp.exp(x - m_new)
    s_new = s_ref[...] * jnp.exp(m_old - m_new) + jnp.sum(y, axis=1, keepdims=True)
    m_ref[...] = m_new
    s_ref[...] = s_new

    @pl.when(j == nb - 1)
    def _():
        lse_ref[...] = m_new + jnp.log(s_new)


def _lse(pred):
    n, c = pred.shape
    nb = pl.cdiv(c, _VB)
    return pl.pallas_call(
        functools.partial(_lse_body, c),
        grid=(nb,),
        in_specs=[pl.BlockSpec((n, _VB), lambda j: (0, j))],
        out_specs=pl.BlockSpec((n, 1), lambda j: (0, 0)),
        out_shape=jax.ShapeDtypeStruct((n, 1), jnp.float32),
        scratch_shapes=[
            pltpu.VMEM((n, 1), jnp.float32),
            pltpu.VMEM((n, 1), jnp.float32),
        ],
    )(pred)


def _final_body(n, k, lse_ref, gat_ref, gt_ref, out_ref):
    nll = lse_ref[...] - gat_ref[...]
    valid = gt_ref[...] != _IGNORE_INDEX
    loss = jnp.where(valid, nll, jnp.float32(0.0))
    total = jnp.sum(loss)

    b = lax.bitcast_convert_type(loss, jnp.int32)
    skey = b ^ (lax.shift_right_arithmetic(b, 31) & jnp.int32(0x7FFFFFFF))
    int_min = jnp.int32(-2147483648)

    def step(i, p):
        cand = p | lax.shift_left(jnp.int32(1), 31 - i)
        cnt = jnp.sum((skey >= (cand ^ int_min)).astype(jnp.int32))
        return jnp.where(cnt >= k, cand, p)

    p = lax.fori_loop(0, 32, step, jnp.int32(0))
    skey_th = p ^ int_min
    cnt_gt = jnp.sum((skey > skey_th).astype(jnp.int32))
    sum_gt = jnp.sum(jnp.where(skey > skey_th, loss, jnp.float32(0.0)))
    bits_th = skey_th ^ (lax.shift_right_arithmetic(skey_th, 31) & jnp.int32(0x7FFFFFFF))
    f_th = lax.bitcast_convert_type(bits_th, jnp.float32)
    topk_sum = sum_gt + (k - cnt_gt).astype(jnp.float32) * f_th

    out = total / jnp.float32(n) + jnp.float32(_TOP_WEIGHT) * topk_sum / jnp.float32(k)
    out_ref[...] = jnp.full((1, 1), out, jnp.float32)


def _finalize(lse, gathered, gt, n, k):
    return pl.pallas_call(
        functools.partial(_final_body, n, k),
        out_shape=jax.ShapeDtypeStruct((1, 1), jnp.float32),
    )(lse, gathered, gt)


def kernel(pred, gt):
    n, c = pred.shape
    k = max(int(_TOP_RATIO * n), 1)
    gathered = _sc_gather(pred.reshape(-1), gt, n, c)
    lse = _lse(pred)
    rows = n // 128
    out = _finalize(
        lse.reshape(rows, 128),
        gathered.reshape(rows, 128),
        gt.reshape(rows, 128),
        n,
        k,
    )
    return out[0, 0]
